# baseline (speedup 1.0000x reference)
import jax
import jax.numpy as jnp
from jax.experimental import pallas as pl
from jax.experimental.pallas import tpu as pltpu

_N_CHUNKS = 8


def _split(h):
    h_hi = h.astype(jnp.bfloat16)
    h_lo = (h - h_hi.astype(jnp.float32)).astype(jnp.bfloat16)
    return h_hi, h_lo


def _stack(h):
    nh = h.shape[0]
    off = -(-nh // 8) * 8
    h_hi, h_lo = _split(h)
    parts = [h_hi]
    if off > nh:
        parts.append(jnp.zeros((off - nh, h.shape[1]), jnp.bfloat16))
    parts.append(h_lo)
    return jnp.concatenate(parts, axis=0), off


def _dg(a, b):
    return jax.lax.dot_general(a, b, (((1,), (0,)), ((), ())),
                               preferred_element_type=jnp.float32)


def _dotd(a, b):
    return jax.lax.dot(a, b)


def _gcn_kernel(adj_hbm, x_hbm, w1t_ref, w2t_ref, w3t_ref,
                b1_ref, b2_ref, b3_ref, lwt_ref, lb_ref, out_ref,
                a_raw, a_bf, x_vmem, sems, x_sem):
    n = adj_hbm.shape[0]
    chunk = n // _N_CHUNKS
    nh = w1t_ref.shape[0]

    copies = [
        pltpu.make_async_copy(
            adj_hbm.at[pl.ds(i * chunk, chunk), :],
            a_raw.at[pl.ds(i * chunk, chunk), :],
            sems.at[i])
        for i in range(_N_CHUNKS)
    ]
    for c in copies:
        c.start()
    x_copy = pltpu.make_async_copy(x_hbm, x_vmem, x_sem)
    x_copy.start()
    x_copy.wait()

    h0 = jax.lax.dot_general(w1t_ref[...], x_vmem[...],
                             (((1,), (1,)), ((), ())),
                             preferred_element_type=jnp.float32)
    hs0, off = _stack(h0)

    acc = jnp.zeros((hs0.shape[0], n), jnp.float32)
    for i in range(_N_CHUNKS):
        copies[i].wait()
        a_i = a_raw[pl.ds(i * chunk, chunk), :].astype(jnp.bfloat16)
        a_bf[pl.ds(i * chunk, chunk), :] = a_i
        lo, hi = i * chunk, (i + 1) * chunk
        acc = acc + _dg(hs0[:, lo:hi], a_i)
    x1 = jnp.maximum(acc[0:nh] + acc[off:off + nh] + b1_ref[...], 0.0)

    def agg(h):
        hs, o = _stack(h)
        y = _dg(hs, a_bf[...])
        return y[0:nh] + y[o:o + nh]

    h1 = _dotd(w2t_ref[...], x1)
    x2 = jnp.maximum(agg(h1) + b2_ref[...], 0.0)
    h2 = _dotd(w3t_ref[...], x2)
    x3 = agg(h2) + b3_ref[...]

    lw = lwt_ref[...]
    lt = (_dotd(lw[:, 0:nh], x1) + _dotd(lw[:, nh:2 * nh], x2)
          + _dotd(lw[:, 2 * nh:3 * nh], x3) + lb_ref[...])

    m = jnp.max(lt, axis=0, keepdims=True)
    s = lt - m
    out_ref[...] = (s - jnp.log(jnp.sum(jnp.exp(s), axis=0, keepdims=True))).T


def kernel(x, edge_index, W1, W2, W3, b1, b2, b3, lin_W, lin_b):
    n = x.shape[0]
    num_classes = lin_W.shape[1]
    vmem = pl.BlockSpec(memory_space=pltpu.MemorySpace.VMEM)
    return pl.pallas_call(
        _gcn_kernel,
        out_shape=jax.ShapeDtypeStruct((n, num_classes), jnp.float32),
        in_specs=[pl.BlockSpec(memory_space=pl.ANY),
                  pl.BlockSpec(memory_space=pl.ANY)] + [vmem] * 8,
        out_specs=vmem,
        scratch_shapes=[
            pltpu.VMEM((n, n), jnp.int32),
            pltpu.VMEM((n, n), jnp.bfloat16),
            pltpu.VMEM(x.shape, jnp.float32),
            pltpu.SemaphoreType.DMA((_N_CHUNKS,)),
            pltpu.SemaphoreType.DMA,
        ],
    )(
        edge_index,
        x,
        W1.T, W2.T, W3.T,
        b1[:, None], b2[:, None], b3[:, None],
        lin_W.T,
        lin_b[:, None],
    )


# speedup vs baseline: 1.0223x; 1.0223x over previous
import jax
import jax.numpy as jnp
from jax.experimental import pallas as pl
from jax.experimental.pallas import tpu as pltpu

_N_CHUNKS = 8


def _split(h):
    h_hi = h.astype(jnp.bfloat16)
    h_lo = (h - h_hi.astype(jnp.float32)).astype(jnp.bfloat16)
    return h_hi, h_lo


def _stack(h):
    nh = h.shape[0]
    off = -(-nh // 8) * 8
    h_hi, h_lo = _split(h)
    parts = [h_hi]
    if off > nh:
        parts.append(jnp.zeros((off - nh, h.shape[1]), jnp.bfloat16))
    parts.append(h_lo)
    return jnp.concatenate(parts, axis=0), off


def _dg(a, b):
    return jax.lax.dot_general(a, b, (((1,), (0,)), ((), ())),
                               preferred_element_type=jnp.float32)


def _dotd(a, b):
    return jax.lax.dot(a, b)


def _gcn_kernel(adj_hbm, x_hbm, w1t_ref, w2t_ref, w3t_ref,
                b1_ref, b2_ref, b3_ref, lwt_ref, lb_ref, out_ref,
                a_raw, a_bf, x_vmem, sems, x_sem):
    n = adj_hbm.shape[0]
    chunk = n // _N_CHUNKS
    nh = w1t_ref.shape[0]

    copies = [
        pltpu.make_async_copy(
            adj_hbm.at[pl.ds(i * chunk, chunk), :],
            a_raw.at[pl.ds(i * chunk, chunk), :],
            sems.at[i])
        for i in range(_N_CHUNKS)
    ]
    x_copy = pltpu.make_async_copy(x_hbm, x_vmem, x_sem)
    x_copy.start()
    for c in copies:
        c.start()
    x_copy.wait()

    h0 = jax.lax.dot_general(w1t_ref[...], x_vmem[...],
                             (((1,), (1,)), ((), ())),
                             preferred_element_type=jnp.float32)
    hs0, off = _stack(h0)

    acc = jnp.zeros((hs0.shape[0], n), jnp.float32)
    for i in range(_N_CHUNKS):
        copies[i].wait()
        a_i = a_raw[pl.ds(i * chunk, chunk), :].astype(jnp.bfloat16)
        a_bf[pl.ds(i * chunk, chunk), :] = a_i
        lo, hi = i * chunk, (i + 1) * chunk
        acc = acc + _dg(hs0[:, lo:hi], a_i)
    x1 = jnp.maximum(acc[0:nh] + acc[off:off + nh] + b1_ref[...], 0.0)

    def agg(h):
        hs, o = _stack(h)
        y = _dg(hs, a_bf[...])
        return y[0:nh] + y[o:o + nh]

    h1 = _dotd(w2t_ref[...], x1)
    x2 = jnp.maximum(agg(h1) + b2_ref[...], 0.0)
    h2 = _dotd(w3t_ref[...], x2)
    x3 = agg(h2) + b3_ref[...]

    lw = lwt_ref[...]
    lt = (_dotd(lw[:, 0:nh], x1) + _dotd(lw[:, nh:2 * nh], x2)
          + _dotd(lw[:, 2 * nh:3 * nh], x3) + lb_ref[...])

    m = jnp.max(lt, axis=0, keepdims=True)
    s = lt - m
    out_ref[...] = (s - jnp.log(jnp.sum(jnp.exp(s), axis=0, keepdims=True))).T


def kernel(x, edge_index, W1, W2, W3, b1, b2, b3, lin_W, lin_b):
    n = x.shape[0]
    num_classes = lin_W.shape[1]
    vmem = pl.BlockSpec(memory_space=pltpu.MemorySpace.VMEM)
    return pl.pallas_call(
        _gcn_kernel,
        out_shape=jax.ShapeDtypeStruct((n, num_classes), jnp.float32),
        in_specs=[pl.BlockSpec(memory_space=pl.ANY),
                  pl.BlockSpec(memory_space=pl.ANY)] + [vmem] * 8,
        out_specs=vmem,
        scratch_shapes=[
            pltpu.VMEM((n, n), jnp.int32),
            pltpu.VMEM((n, n), jnp.bfloat16),
            pltpu.VMEM(x.shape, jnp.float32),
            pltpu.SemaphoreType.DMA((_N_CHUNKS,)),
            pltpu.SemaphoreType.DMA,
        ],
    )(
        edge_index,
        x,
        W1.T, W2.T, W3.T,
        b1[:, None], b2[:, None], b3[:, None],
        lin_W.T,
        lin_b[:, None],
    )


# speedup vs baseline: 1.0814x; 1.0579x over previous
import jax
import jax.numpy as jnp
from jax.experimental import pallas as pl
from jax.experimental.pallas import tpu as pltpu

_N_CHUNKS = 8


def _split(h):
    h_hi = h.astype(jnp.bfloat16)
    h_lo = (h - h_hi.astype(jnp.float32)).astype(jnp.bfloat16)
    return h_hi, h_lo


def _stack(h):
    nh = h.shape[0]
    off = -(-nh // 8) * 8
    h_hi, h_lo = _split(h)
    parts = [h_hi]
    if off > nh:
        parts.append(jnp.zeros((off - nh, h.shape[1]), jnp.bfloat16))
    parts.append(h_lo)
    return jnp.concatenate(parts, axis=0), off


def _dg(a, b):
    return jax.lax.dot_general(a, b, (((1,), (0,)), ((), ())),
                               preferred_element_type=jnp.float32)


def _dotd(a, b):
    return jax.lax.dot(a, b)


def _gcn_kernel(adj_hbm, x_ref, w1t_ref, w2t_ref, w3t_ref,
                b1_ref, b2_ref, b3_ref, lwt_ref, lb_ref, out_ref,
                a_raw, a_bf, sems):
    n = adj_hbm.shape[0]
    chunk = n // _N_CHUNKS
    nh = w1t_ref.shape[0]

    copies = [
        pltpu.make_async_copy(
            adj_hbm.at[pl.ds(i * chunk, chunk), :],
            a_raw.at[pl.ds(i * chunk, chunk), :],
            sems.at[i])
        for i in range(_N_CHUNKS)
    ]
    for c in copies:
        c.start()

    h0 = jax.lax.dot_general(w1t_ref[...], x_ref[...],
                             (((1,), (1,)), ((), ())),
                             preferred_element_type=jnp.float32)
    hs0, off = _stack(h0)

    acc = jnp.zeros((hs0.shape[0], n), jnp.float32)
    for i in range(_N_CHUNKS):
        copies[i].wait()
        a_i = a_raw[pl.ds(i * chunk, chunk), :].astype(jnp.bfloat16)
        a_bf[pl.ds(i * chunk, chunk), :] = a_i
        lo, hi = i * chunk, (i + 1) * chunk
        acc = acc + _dg(hs0[:, lo:hi], a_i)
    x1 = jnp.maximum(acc[0:nh] + acc[off:off + nh] + b1_ref[...], 0.0)

    def agg(h):
        hs, o = _stack(h)
        y = _dg(hs, a_bf[...])
        return y[0:nh] + y[o:o + nh]

    h1 = _dotd(w2t_ref[...], x1)
    x2 = jnp.maximum(agg(h1) + b2_ref[...], 0.0)
    h2 = _dotd(w3t_ref[...], x2)
    x3 = agg(h2) + b3_ref[...]

    lw = lwt_ref[...]
    lt = (_dotd(lw[:, 0:nh], x1) + _dotd(lw[:, nh:2 * nh], x2)
          + _dotd(lw[:, 2 * nh:3 * nh], x3) + lb_ref[...])

    m = jnp.max(lt, axis=0, keepdims=True)
    s = lt - m
    out_ref[...] = (s - jnp.log(jnp.sum(jnp.exp(s), axis=0, keepdims=True))).T


def kernel(x, edge_index, W1, W2, W3, b1, b2, b3, lin_W, lin_b):
    n = x.shape[0]
    num_classes = lin_W.shape[1]
    vmem = pl.BlockSpec(memory_space=pltpu.MemorySpace.VMEM)
    return pl.pallas_call(
        _gcn_kernel,
        out_shape=jax.ShapeDtypeStruct((n, num_classes), jnp.float32),
        in_specs=[pl.BlockSpec(memory_space=pl.ANY)] + [vmem] * 9,
        out_specs=vmem,
        scratch_shapes=[
            pltpu.VMEM((n, n), jnp.int32),
            pltpu.VMEM((n, n), jnp.bfloat16),
            pltpu.SemaphoreType.DMA((_N_CHUNKS,)),
        ],
    )(
        edge_index,
        x,
        W1.T, W2.T, W3.T,
        b1[:, None], b2[:, None], b3[:, None],
        lin_W.T,
        lin_b[:, None],
    )


# speedup vs baseline: 1.0851x; 1.0034x over previous
import jax
import jax.numpy as jnp
from jax.experimental import pallas as pl
from jax.experimental.pallas import tpu as pltpu

_N_CHUNKS = 16


def _split(h):
    h_hi = h.astype(jnp.bfloat16)
    h_lo = (h - h_hi.astype(jnp.float32)).astype(jnp.bfloat16)
    return h_hi, h_lo


def _stack(h):
    nh = h.shape[0]
    off = -(-nh // 8) * 8
    h_hi, h_lo = _split(h)
    parts = [h_hi]
    if off > nh:
        parts.append(jnp.zeros((off - nh, h.shape[1]), jnp.bfloat16))
    parts.append(h_lo)
    return jnp.concatenate(parts, axis=0), off


def _dg(a, b):
    return jax.lax.dot_general(a, b, (((1,), (0,)), ((), ())),
                               preferred_element_type=jnp.float32)


def _dotd(a, b):
    return jax.lax.dot(a, b)


def _gcn_kernel(adj_hbm, x_ref, w1t_ref, w2t_ref, w3t_ref,
                b1_ref, b2_ref, b3_ref, lwt_ref, lb_ref, out_ref,
                a_raw, a_bf, sems):
    n = adj_hbm.shape[0]
    chunk = n // _N_CHUNKS
    nh = w1t_ref.shape[0]

    copies = [
        pltpu.make_async_copy(
            adj_hbm.at[pl.ds(i * chunk, chunk), :],
            a_raw.at[pl.ds(i * chunk, chunk), :],
            sems.at[i])
        for i in range(_N_CHUNKS)
    ]
    for c in copies:
        c.start()

    h0 = jax.lax.dot_general(w1t_ref[...], x_ref[...],
                             (((1,), (1,)), ((), ())),
                             preferred_element_type=jnp.float32)
    hs0, off = _stack(h0)

    acc = jnp.zeros((hs0.shape[0], n), jnp.float32)
    for i in range(_N_CHUNKS):
        copies[i].wait()
        a_i = a_raw[pl.ds(i * chunk, chunk), :].astype(jnp.bfloat16)
        a_bf[pl.ds(i * chunk, chunk), :] = a_i
        lo, hi = i * chunk, (i + 1) * chunk
        acc = acc + _dg(hs0[:, lo:hi], a_i)
    x1 = jnp.maximum(acc[0:nh] + acc[off:off + nh] + b1_ref[...], 0.0)

    def agg(h):
        hs, o = _stack(h)
        y = _dg(hs, a_bf[...])
        return y[0:nh] + y[o:o + nh]

    h1 = _dotd(w2t_ref[...], x1)
    x2 = jnp.maximum(agg(h1) + b2_ref[...], 0.0)
    h2 = _dotd(w3t_ref[...], x2)
    x3 = agg(h2) + b3_ref[...]

    lw = lwt_ref[...]
    lt = (_dotd(lw[:, 0:nh], x1) + _dotd(lw[:, nh:2 * nh], x2)
          + _dotd(lw[:, 2 * nh:3 * nh], x3) + lb_ref[...])

    m = jnp.max(lt, axis=0, keepdims=True)
    s = lt - m
    out_ref[...] = (s - jnp.log(jnp.sum(jnp.exp(s), axis=0, keepdims=True))).T


def kernel(x, edge_index, W1, W2, W3, b1, b2, b3, lin_W, lin_b):
    n = x.shape[0]
    num_classes = lin_W.shape[1]
    vmem = pl.BlockSpec(memory_space=pltpu.MemorySpace.VMEM)
    return pl.pallas_call(
        _gcn_kernel,
        out_shape=jax.ShapeDtypeStruct((n, num_classes), jnp.float32),
        in_specs=[pl.BlockSpec(memory_space=pl.ANY)] + [vmem] * 9,
        out_specs=vmem,
        scratch_shapes=[
            pltpu.VMEM((n, n), jnp.int32),
            pltpu.VMEM((n, n), jnp.bfloat16),
            pltpu.SemaphoreType.DMA((_N_CHUNKS,)),
        ],
    )(
        edge_index,
        x,
        W1.T, W2.T, W3.T,
        b1[:, None], b2[:, None], b3[:, None],
        lin_W.T,
        lin_b[:, None],
    )
